# baseline (speedup 1.0000x reference)
import functools

import jax
import jax.numpy as jnp
from jax import lax
from jax.experimental import pallas as pl
from jax.experimental.pallas import tpu as pltpu
from jax.experimental.pallas import tpu_sc as plsc

HIDDEN = 3072
SEQ = 4096
NUM_TOKENS = 2 * SEQ
NC = 2
NS = 16
NW = NC * NS
PER_W = NUM_TOKENS // NW
CHUNK = 16
SIZES = [8, 8] + [16] * 14 + [8, 8]
OFFS = [sum(SIZES[:k]) for k in range(len(SIZES))]
K = len(SIZES)
LOOP_LO, LOOP_HI = 2, 14
NBUF = 2

_mesh = plsc.VectorSubcoreMesh(core_axis_name="c", subcore_axis_name="s")


@functools.partial(
    pl.kernel,
    mesh=_mesh,
    out_type=jax.ShapeDtypeStruct((2, SEQ, HIDDEN), jnp.float32),
    scratch_types=[
        pltpu.VMEM((PER_W,), jnp.int32),
        pltpu.VMEM((NBUF, CHUNK, HIDDEN), jnp.float32),
        pltpu.SemaphoreType.DMA,
        pltpu.SemaphoreType.DMA,
        pltpu.SemaphoreType.DMA,
        pltpu.SemaphoreType.DMA,
    ],
)
def _embed_lookup(table_hbm, ids_hbm, out_hbm, idx_v, rows_v, si0, si1, so0, so1):
    in_sem = (si0, si1)
    out_sem = (so0, so1)
    wid = lax.axis_index("s") * NC + lax.axis_index("c")
    tiles_per_row = SEQ // PER_W
    brow = wid // tiles_per_row
    bcol = (wid % tiles_per_row) * PER_W
    pltpu.sync_copy(ids_hbm.at[brow, pl.ds(bcol, PER_W)], idx_v)

    def gather(j, b):
        return pltpu.async_copy(
            table_hbm.at[idx_v.at[pl.ds(OFFS[j], SIZES[j])]],
            rows_v.at[b, pl.ds(0, SIZES[j])],
            in_sem[b],
        )

    def put(j, b):
        return pltpu.async_copy(
            rows_v.at[b, pl.ds(0, SIZES[j])],
            out_hbm.at[brow, pl.ds(bcol + OFFS[j], SIZES[j])],
            out_sem[b],
        )

    def gather_dyn(off, b):
        return pltpu.async_copy(
            table_hbm.at[idx_v.at[pl.ds(off, CHUNK)]],
            rows_v.at[b, pl.ds(0, CHUNK)],
            in_sem[b],
        )

    def put_dyn(off, b):
        return pltpu.async_copy(
            rows_v.at[b, pl.ds(0, CHUNK)],
            out_hbm.at[brow, pl.ds(bcol + off, CHUNK)],
            out_sem[b],
        )

    def wait_gather16(b):
        pltpu.make_async_copy(
            table_hbm.at[pl.ds(0, CHUNK)], rows_v.at[b, pl.ds(0, CHUNK)], in_sem[b]
        ).wait()

    def wait_put16(b):
        pltpu.make_async_copy(
            rows_v.at[b, pl.ds(0, CHUNK)],
            out_hbm.at[brow, pl.ds(bcol, CHUNK)],
            out_sem[b],
        ).wait()

    gcp = [gather(0, 0), gather(1, 1)]
    for j in (0, 1):
        gcp[j].wait()
        put(j, j).wait()
        gather_dyn(OFFS[LOOP_LO] + (j * CHUNK), j)

    @pl.loop(0, (LOOP_HI - LOOP_LO) // NBUF)
    def _middle(g):
        off = OFFS[LOOP_LO] + g * (NBUF * CHUNK)
        for b in range(NBUF):
            wait_gather16(b)
            put_dyn(off + b * CHUNK, b)
            wait_put16(b)
            gather_dyn(off + (NBUF + b) * CHUNK, b)

    for j in (LOOP_HI, LOOP_HI + 1):
        b = j % NBUF
        wait_gather16(b)
        put(j, b).wait()
        gcp[b] = gather(j + NBUF, b)
    pcp = [None, None]
    for j in (K - 2, K - 1):
        b = j % NBUF
        gcp[b].wait()
        pcp[b] = put(j, b)
    pcp[0].wait()
    pcp[1].wait()


def kernel(embed_weight, input_ids):
    return _embed_lookup(embed_weight, input_ids.astype(jnp.int32))


# speedup vs baseline: 1.0073x; 1.0073x over previous
import functools

import jax
import jax.numpy as jnp
from jax import lax
from jax.experimental import pallas as pl
from jax.experimental.pallas import tpu as pltpu
from jax.experimental.pallas import tpu_sc as plsc

HIDDEN = 3072
SEQ = 4096
NUM_TOKENS = 2 * SEQ
NC = 2
NS = 16
NW = NC * NS
PER_W = NUM_TOKENS // NW
CHUNK = 16
SIZES = [8] + [16] * 15 + [8]
OFFS = [sum(SIZES[:k]) for k in range(len(SIZES))]
NBUF = 2

_mesh = plsc.VectorSubcoreMesh(core_axis_name="c", subcore_axis_name="s")


@functools.partial(
    pl.kernel,
    mesh=_mesh,
    out_type=jax.ShapeDtypeStruct((2, SEQ, HIDDEN), jnp.float32),
    scratch_types=[
        pltpu.VMEM((PER_W,), jnp.int32),
        pltpu.VMEM((NBUF, CHUNK, HIDDEN), jnp.float32),
        pltpu.SemaphoreType.DMA,
        pltpu.SemaphoreType.DMA,
        pltpu.SemaphoreType.DMA,
        pltpu.SemaphoreType.DMA,
    ],
)
def _embed_lookup(table_hbm, ids_hbm, out_hbm, idx_v, rows_v, si0, si1, so0, so1):
    in_sem = (si0, si1)
    out_sem = (so0, so1)
    wid = lax.axis_index("s") * NC + lax.axis_index("c")
    tiles_per_row = SEQ // PER_W
    brow = wid // tiles_per_row
    bcol = (wid % tiles_per_row) * PER_W
    pltpu.sync_copy(ids_hbm.at[brow, pl.ds(bcol, PER_W)], idx_v)

    def gather(j, b):
        return pltpu.async_copy(
            table_hbm.at[idx_v.at[pl.ds(OFFS[j], SIZES[j])]],
            rows_v.at[b, pl.ds(0, SIZES[j])],
            in_sem[b],
        )

    def put(j, b):
        return pltpu.async_copy(
            rows_v.at[b, pl.ds(0, SIZES[j])],
            out_hbm.at[brow, pl.ds(bcol + OFFS[j], SIZES[j])],
            out_sem[b],
        )

    nchunk = len(SIZES)
    gcp = [gather(0, 0), gather(1, 1)]
    pcp = [None, None]
    for j in range(nchunk):
        b = j % NBUF
        gcp[b].wait()
        pcp[b] = put(j, b)
        if j + NBUF < nchunk:
            pcp[b].wait()
            gcp[b] = gather(j + NBUF, b)
    pcp[0].wait()
    pcp[1].wait()


def kernel(embed_weight, input_ids):
    return _embed_lookup(embed_weight, input_ids.astype(jnp.int32))
